# baseline (speedup 1.0000x reference)
import functools

import jax
import jax.numpy as jnp
from jax import lax
from jax.experimental import pallas as pl
from jax.experimental.pallas import tpu as pltpu
from jax.experimental.pallas import tpu_sc as plsc

EMBED_D = 32
PACK = 128 // EMBED_D
B_TOTAL = 4096 * 50
NUM_CORES = 2
NUM_SUBCORES = 16
NW = NUM_CORES * NUM_SUBCORES
B_PER_W = B_TOTAL // NW
CHUNK = 1600
N_CHUNKS = B_PER_W // CHUNK
LANES = 16

_mesh = plsc.VectorSubcoreMesh(core_axis_name="c", subcore_axis_name="s")


@functools.partial(
    pl.kernel,
    mesh=_mesh,
    out_type=jax.ShapeDtypeStruct((B_TOTAL, EMBED_D), jnp.float32),
    scratch_types=[
        pltpu.VMEM((B_PER_W,), jnp.int32),
        pltpu.VMEM((CHUNK, EMBED_D), jnp.float32),
        pltpu.VMEM((CHUNK, EMBED_D), jnp.float32),
        pltpu.SemaphoreType.DMA,
        pltpu.SemaphoreType.DMA,
        pltpu.SemaphoreType.DMA,
    ],
    compiler_params=pltpu.CompilerParams(
        use_tc_tiling_on_sc=False, needs_layout_passes=False),
)
def _gather_kernel(idx_hbm, tab_hbm, out_hbm, idx_v, buf_a, buf_b,
                   g_sem, wa_sem, wb_sem):
    wid = lax.axis_index("s") * NUM_CORES + lax.axis_index("c")
    base = wid * B_PER_W
    pltpu.sync_copy(idx_hbm.at[pl.ds(base, B_PER_W)], idx_v)

    bufs = (buf_a, buf_b)
    wsem = (wa_sem, wb_sem)

    def fill(c, buf):
        def m_body(m, _):
            vec = idx_v[pl.ds(c * CHUNK + m * LANES, LANES)]
            for l in range(LANES):
                r = vec[l]
                g = lax.shift_right_logical(r, 2)
                c0 = (r & (PACK - 1)) * EMBED_D
                pltpu.async_copy(
                    tab_hbm.at[pl.ds(g, 1), pl.ds(c0, EMBED_D)],
                    buf.at[pl.ds(m * LANES + l, 1)], g_sem)
            return _

        lax.fori_loop(0, CHUNK // LANES, m_body, 0)
        pltpu.make_async_copy(tab_hbm.at[pl.ds(0, CHUNK), pl.ds(0, EMBED_D)],
                              buf, g_sem).wait()

    writes = [None, None]
    for c in range(N_CHUNKS):
        p = c % 2
        if writes[p] is not None:
            writes[p].wait()
        fill(c, bufs[p])
        writes[p] = pltpu.async_copy(
            bufs[p], out_hbm.at[pl.ds(base + c * CHUNK, CHUNK)], wsem[p])
    for w in writes:
        if w is not None:
            w.wait()


def kernel(x, wordmat):
    idx = x.reshape(-1).astype(jnp.int32)
    tab = wordmat.reshape(wordmat.shape[0] // PACK, 128)
    out = _gather_kernel(idx, tab)
    return out.reshape(x.shape + (EMBED_D,))
